# baseline (speedup 1.0000x reference)
import functools

import jax
import jax.numpy as jnp
from jax import lax
from jax.experimental import pallas as pl
from jax.experimental.pallas import tpu as pltpu
from jax.experimental.pallas import tpu_sc as plsc

NC = 2
NS = 16
K = 128
NB = 2
IDXB = 16


def _sc_agg_kernel(blocks0, blocks1, rows_per_tile, n_pad, dh,
                   src_hbm, dst_hbm, x0_hbm, x1_hbm, out0_hbm, out1_hbm,
                   src_v, dst_v, rows_v, shm, gsems):
    cid = lax.axis_index("c")
    sid = lax.axis_index("s")
    on0 = cid == 0
    nblocks = jnp.where(on0, blocks0, blocks1)
    base_chunk = IDXB * jnp.where(on0, sid * blocks0,
                                  NS * blocks0 + sid * blocks1)

    xbase = NS * rows_per_tile
    zv = jnp.zeros((16,), jnp.float32)
    nrt = n_pad // NS
    for xp_hbm, outp_hbm in ((x0_hbm, out0_hbm), (x1_hbm, out1_hbm)):
        def zero_body(i, carry):
            for jj in range(dh // 16):
                rows_v[0][i, pl.ds(jj * 16, 16)] = zv
            return carry

        lax.fori_loop(0, K, zero_body, 0)
        for c in range(rows_per_tile // K):
            pltpu.sync_copy(rows_v[0],
                            shm.at[pl.ds(sid * rows_per_tile + c * K, K)])
        for c in range(nrt // K):
            pltpu.sync_copy(xp_hbm.at[pl.ds(sid * nrt + c * K, K)],
                            rows_v[1])
            pltpu.sync_copy(rows_v[1],
                            shm.at[pl.ds(xbase + sid * nrt + c * K, K)])
        plsc.subcore_barrier()

        def body(bi, carry):
            cbase = base_chunk + bi * IDXB
            pltpu.sync_copy(src_hbm.at[pl.ds(cbase, IDXB)], src_v)
            pltpu.sync_copy(dst_hbm.at[pl.ds(cbase, IDXB)], dst_v)
            for b in range(NB):
                pltpu.async_copy(shm.at[src_v.at[b]], rows_v[b], gsems[b])
            for j in range(IDXB):
                b = j % NB
                pltpu.make_async_copy(shm.at[pl.ds(0, K)], rows_v[b],
                                      gsems[b]).wait()
                pltpu.sync_copy(rows_v[b], shm.at[dst_v.at[j]], add=True)
                if j + NB < IDXB:
                    pltpu.async_copy(shm.at[src_v.at[j + NB]], rows_v[b],
                                     gsems[b])
            return carry

        lax.fori_loop(0, nblocks, body, 0)
        plsc.subcore_barrier()

        r0 = sid * rows_per_tile
        pltpu.sync_copy(shm.at[pl.ds(r0, rows_per_tile)],
                        outp_hbm.at[cid, pl.ds(r0, rows_per_tile)])


def _sc_aggregate(src, dst, x0, x1, acc_rows, rows_per_tile,
                  blocks0, blocks1):
    n_pad, dh = x0.shape
    mesh = plsc.VectorSubcoreMesh(core_axis_name="c", subcore_axis_name="s")
    kern = pl.kernel(
        functools.partial(_sc_agg_kernel, blocks0, blocks1, rows_per_tile,
                          n_pad, dh),
        out_type=[jax.ShapeDtypeStruct((NC, acc_rows, dh), jnp.float32),
                  jax.ShapeDtypeStruct((NC, acc_rows, dh), jnp.float32)],
        mesh=mesh,
        scratch_types=[
            pltpu.VMEM((IDXB, K), jnp.int32),
            pltpu.VMEM((IDXB, K), jnp.int32),
            [pltpu.VMEM((K, dh), jnp.float32) for _ in range(NB)],
            pltpu.VMEM_SHARED((acc_rows + n_pad, dh), jnp.float32),
            [pltpu.SemaphoreType.DMA for _ in range(NB)],
        ],
        compiler_params=pltpu.CompilerParams(use_tc_tiling_on_sc=False),
    )
    return kern(src, dst, x0, x1)


def _mlp_body(x_ref, a00_ref, a01_ref, a10_ref, a11_ref,
              w1_ref, b1_ref, w2_ref, b2_ref, wc_ref, bc_ref, o_ref):
    agg = jnp.concatenate(
        [a00_ref[0] + a01_ref[0], a10_ref[0] + a11_ref[0]], axis=1)
    h = x_ref[...] + agg
    h = jnp.maximum(
        jnp.dot(h, w1_ref[...], preferred_element_type=jnp.float32)
        + b1_ref[...], 0.0)
    h = jnp.dot(h, w2_ref[...], preferred_element_type=jnp.float32) + b2_ref[...]
    o_ref[...] = (
        jnp.dot(jnp.maximum(h, 0.0), wc_ref[...],
                preferred_element_type=jnp.float32) + bc_ref[...])


def _mlp(x, p0, p1, W1, b1, W2, b2, Wc, bc, blk):
    n, d = x.shape
    dh = d // 2
    d_out = Wc.shape[1]
    grid = n // blk
    w_spec = pl.BlockSpec((d, d), lambda i: (0, 0))
    b_spec = pl.BlockSpec((1, d), lambda i: (0, 0))
    half0 = pl.BlockSpec((1, blk, dh), lambda i: (0, i, 0))
    half1 = pl.BlockSpec((1, blk, dh), lambda i: (1, i, 0))
    return pl.pallas_call(
        _mlp_body,
        grid=(grid,),
        in_specs=[
            pl.BlockSpec((blk, d), lambda i: (i, 0)),
            half0, half1, half0, half1,
            w_spec, b_spec, w_spec, b_spec, w_spec,
            pl.BlockSpec((1, d_out), lambda i: (0, 0)),
        ],
        out_specs=pl.BlockSpec((blk, d_out), lambda i: (i, 0)),
        out_shape=jax.ShapeDtypeStruct((n, d_out), jnp.float32),
    )(x, p0, p0, p1, p1, W1, b1.reshape(1, -1), W2, b2.reshape(1, -1),
      Wc, bc.reshape(1, -1))


def kernel(x, edge_index, W1, b1, W2, b2, Wc, bc):
    n, d = x.shape
    dh = d // 2
    e = edge_index.shape[1]

    total_blocks = NC * NS * (-(-e // (NC * NS * K * IDXB)))
    per_tile_blocks = total_blocks // NS
    blocks0 = per_tile_blocks // 2
    blocks1 = per_tile_blocks - blocks0
    rows_per_tile = K * (-(-(n + 1) // (NS * K)))
    acc_rows = NS * rows_per_tile

    e_pad = total_blocks * IDXB * K - e
    src = edge_index[0].astype(jnp.int32) + acc_rows
    dst = edge_index[1].astype(jnp.int32)
    if e_pad:
        src = jnp.concatenate([src, jnp.full((e_pad,), acc_rows, jnp.int32)])
        dst = jnp.concatenate([dst, jnp.full((e_pad,), n, jnp.int32)])
    src = src.reshape(total_blocks * IDXB, K)
    dst = dst.reshape(total_blocks * IDXB, K)

    n_pad = K * NS * (-(-n // (K * NS)))
    x0 = x[:, :dh]
    x1 = x[:, dh:]
    if n_pad != n:
        x0 = jnp.pad(x0, ((0, n_pad - n), (0, 0)))
        x1 = jnp.pad(x1, ((0, n_pad - n), (0, 0)))

    p0, p1 = _sc_aggregate(src, dst, x0, x1, acc_rows, rows_per_tile,
                           blocks0, blocks1)

    blk = 2000 if n % 2000 == 0 else (1000 if n % 1000 == 0 else 8)
    return _mlp(x, p0, p1, W1, b1, W2, b2, Wc, bc, blk)
